# baseline (speedup 1.0000x reference)
import functools

import jax
import jax.numpy as jnp
from jax import lax
from jax.experimental import pallas as pl
from jax.experimental.pallas import tpu as pltpu
from jax.experimental.pallas import tpu_sc as plsc

N_NODES = 10000
N_PAD = 10240
N_EDGES = 320000
D = 128

NC = 2
NS = 16
NW = NC * NS
EPW = N_EDGES // NW
C = 80
NCHUNK = EPW // C
RSTRIPE = N_PAD // NS
ZR = ZW_ROWS = 8
ZW = ZR * D


def _proj_nodes(x, w1a, w1b):
    bn = 2000

    def body(x_ref, wa_ref, wb_ref, oa_ref, ob_ref):
        xv = x_ref[...]
        oa_ref[...] = jnp.dot(xv, wa_ref[...], preferred_element_type=jnp.float32)
        ob_ref[...] = jnp.dot(xv, wb_ref[...], preferred_element_type=jnp.float32)

    return pl.pallas_call(
        body,
        grid=(N_NODES // bn,),
        in_specs=[
            pl.BlockSpec((bn, D), lambda i: (i, 0)),
            pl.BlockSpec((D, D), lambda i: (0, 0)),
            pl.BlockSpec((D, D), lambda i: (0, 0)),
        ],
        out_specs=[
            pl.BlockSpec((bn, D), lambda i: (i, 0)),
            pl.BlockSpec((bn, D), lambda i: (i, 0)),
        ],
        out_shape=[
            jax.ShapeDtypeStruct((N_NODES, D), jnp.float32),
            jax.ShapeDtypeStruct((N_NODES, D), jnp.float32),
        ],
    )(x, w1a, w1b)


def _proj_edges(edge_attr, w1c, b1):
    be = 8000

    def body(e_ref, w_ref, b_ref, o_ref):
        o_ref[...] = (
            jnp.dot(e_ref[...], w_ref[...], preferred_element_type=jnp.float32)
            + b_ref[...]
        )

    return pl.pallas_call(
        body,
        grid=(N_EDGES // be,),
        in_specs=[
            pl.BlockSpec((be, 16), lambda i: (i, 0)),
            pl.BlockSpec((16, D), lambda i: (0, 0)),
            pl.BlockSpec((1, D), lambda i: (0, 0)),
        ],
        out_specs=pl.BlockSpec((be, D), lambda i: (i, 0)),
        out_shape=jax.ShapeDtypeStruct((N_EDGES, D), jnp.float32),
    )(edge_attr, w1c, b1.reshape(1, D))


def _final_mm(s_parts, w2):
    bn = 2048

    def body(s_ref, w_ref, o_ref):
        s = s_ref[0] + s_ref[1]
        o_ref[...] = jnp.dot(s, w_ref[...], preferred_element_type=jnp.float32)

    return pl.pallas_call(
        body,
        grid=(N_PAD // bn,),
        in_specs=[
            pl.BlockSpec((2, bn, D), lambda i: (0, i, 0)),
            pl.BlockSpec((D, D), lambda i: (0, 0)),
        ],
        out_specs=pl.BlockSpec((bn, D), lambda i: (i, 0)),
        out_shape=jax.ShapeDtypeStruct((N_PAD, D), jnp.float32),
    )(s_parts, w2)


@functools.partial(
    pl.kernel,
    out_type=jax.ShapeDtypeStruct((NC, N_PAD, D), jnp.float32),
    mesh=plsc.VectorSubcoreMesh(core_axis_name="c", subcore_axis_name="s"),
    scratch_types=[
        pltpu.VMEM((C,), jnp.int32),
        pltpu.VMEM((C,), jnp.int32),
        pltpu.VMEM((C, D), jnp.float32),
        pltpu.VMEM((C, D), jnp.float32),
        pltpu.VMEM((C, D), jnp.float32),
        pltpu.VMEM((D,), jnp.float32),
        pltpu.VMEM((ZR, D), jnp.float32),
        pltpu.VMEM_SHARED((N_PAD, D), jnp.float32),
        pltpu.SemaphoreType.DMA,
        pltpu.SemaphoreType.DMA,
        pltpu.SemaphoreType.DMA,
    ],
)
def _sc_edge_kernel(xa_hbm, xb_hbm, ec_hbm, row_hbm, col_hbm,
                    t_hbm, out_hbm,
                    rowv, colv, bufa, bufb, bufe, tbuf, zbuf,
                    s_acc, sem_a, sem_b, sem_e):
    cid = lax.axis_index("c")
    sid = lax.axis_index("s")
    wid = sid * NC + cid

    zvec = jnp.zeros((16,), jnp.float32)
    iota16 = lax.broadcasted_iota(jnp.int32, (16,), 0)

    pltpu.sync_copy(t_hbm, tbuf)
    tvs = [tbuf[pl.ds(u * 16, 16)] for u in range(D // 16)]

    for zr in range(ZR):
        for u in range(D // 16):
            zbuf[zr, pl.ds(u * 16, 16)] = zvec

    def zero_stripe(k, _):
        pltpu.sync_copy(zbuf, s_acc.at[pl.ds(sid * RSTRIPE + k * ZR, ZR)])
        return 0
    lax.fori_loop(0, RSTRIPE // ZR, zero_stripe, 0)

    plsc.subcore_barrier()

    base_e = wid * EPW

    def chunk(i, _):
        off = base_e + i * C
        pltpu.sync_copy(row_hbm.at[pl.ds(off, C)], rowv)
        pltpu.sync_copy(col_hbm.at[pl.ds(off, C)], colv)
        da = pltpu.async_copy(xa_hbm.at[rowv], bufa, sem_a)
        db = pltpu.async_copy(xb_hbm.at[colv], bufb, sem_b)
        de = pltpu.async_copy(ec_hbm.at[pl.ds(off, C)], bufe, sem_e)
        da.wait()
        db.wait()
        de.wait()

        def comp(r, _):
            for u in range(D // 16):
                cc = u * 16
                v = bufa[r, pl.ds(cc, 16)] + bufb[r, pl.ds(cc, 16)] \
                    + bufe[r, pl.ds(cc, 16)]
                bufe[r, pl.ds(cc, 16)] = (
                    jnp.maximum(v, jnp.float32(0.0)) + tvs[u])
            return 0
        lax.fori_loop(0, C, comp, 0)

        plsc.subcore_barrier()

        def scat_ser(k, _):
            @pl.when(sid == k)
            def _():
                pltpu.sync_copy(bufe, s_acc.at[rowv], add=True)
            plsc.subcore_barrier()
            return 0
        lax.fori_loop(0, NS, scat_ser, 0)
        return 0

    lax.fori_loop(0, NCHUNK, chunk, 0)

    plsc.subcore_barrier()

    def outcp(k, _):
        r0 = sid * RSTRIPE + k * C
        pltpu.sync_copy(s_acc.at[pl.ds(r0, C)], bufa)
        pltpu.sync_copy(bufa, out_hbm.at[cid, pl.ds(r0, C)])
        return 0
    lax.fori_loop(0, RSTRIPE // C, outcp, 0)


def kernel(x, edge_index, edge_attr, W1, b1, W2, b2):
    x = x.astype(jnp.float32)
    row = edge_index[0].astype(jnp.int32)
    col = edge_index[1].astype(jnp.int32)

    w1a = W1[:D]
    w1b = W1[D:2 * D]
    w1c = W1[2 * D:]

    t = jnp.linalg.solve(W2.T, b2)
    t = t + jnp.linalg.solve(W2.T, b2 - t @ W2)

    xa, xb = _proj_nodes(x, w1a, w1b)
    ec = _proj_edges(edge_attr, w1c, b1)

    s_parts = _sc_edge_kernel(xa, xb, ec, row, col, t)

    return _final_mm(s_parts, W2)[:N_NODES]


# speedup vs baseline: 1.1674x; 1.1674x over previous
import functools

import jax
import jax.numpy as jnp
from jax import lax
from jax.experimental import pallas as pl
from jax.experimental.pallas import tpu as pltpu
from jax.experimental.pallas import tpu_sc as plsc

N_NODES = 10000
N_PAD = 10240
N_EDGES = 320000
D = 128

NC = 2
NS = 16
NW = NC * NS
EPW = N_EDGES // NW
C = 80
NCHUNK = EPW // C
RSTRIPE = N_PAD // NS
ZR = ZW_ROWS = 8
ZW = ZR * D


def _proj_nodes(x, w1a, w1b):
    bn = 2000

    def body(x_ref, wa_ref, wb_ref, oa_ref, ob_ref):
        xv = x_ref[...]
        oa_ref[...] = jnp.dot(xv, wa_ref[...], preferred_element_type=jnp.float32)
        ob_ref[...] = jnp.dot(xv, wb_ref[...], preferred_element_type=jnp.float32)

    return pl.pallas_call(
        body,
        grid=(N_NODES // bn,),
        in_specs=[
            pl.BlockSpec((bn, D), lambda i: (i, 0)),
            pl.BlockSpec((D, D), lambda i: (0, 0)),
            pl.BlockSpec((D, D), lambda i: (0, 0)),
        ],
        out_specs=[
            pl.BlockSpec((bn, D), lambda i: (i, 0)),
            pl.BlockSpec((bn, D), lambda i: (i, 0)),
        ],
        out_shape=[
            jax.ShapeDtypeStruct((N_NODES, D), jnp.float32),
            jax.ShapeDtypeStruct((N_NODES, D), jnp.float32),
        ],
    )(x, w1a, w1b)


def _proj_edges(edge_attr, w1c, b1):
    be = 8000

    def body(e_ref, w_ref, b_ref, o_ref):
        o_ref[...] = (
            jnp.dot(e_ref[...], w_ref[...], preferred_element_type=jnp.float32)
            + b_ref[...]
        )

    return pl.pallas_call(
        body,
        grid=(N_EDGES // be,),
        in_specs=[
            pl.BlockSpec((be, 16), lambda i: (i, 0)),
            pl.BlockSpec((16, D), lambda i: (0, 0)),
            pl.BlockSpec((1, D), lambda i: (0, 0)),
        ],
        out_specs=pl.BlockSpec((be, D), lambda i: (i, 0)),
        out_shape=jax.ShapeDtypeStruct((N_EDGES, D), jnp.float32),
    )(edge_attr, w1c, b1.reshape(1, D))


def _final_mm(s_parts, w2):
    bn = 2048

    def body(s_ref, w_ref, o_ref):
        s = s_ref[0] + s_ref[1]
        o_ref[...] = jnp.dot(s, w_ref[...], preferred_element_type=jnp.float32)

    return pl.pallas_call(
        body,
        grid=(N_PAD // bn,),
        in_specs=[
            pl.BlockSpec((2, bn, D), lambda i: (0, i, 0)),
            pl.BlockSpec((D, D), lambda i: (0, 0)),
        ],
        out_specs=pl.BlockSpec((bn, D), lambda i: (i, 0)),
        out_shape=jax.ShapeDtypeStruct((N_PAD, D), jnp.float32),
    )(s_parts, w2)


@functools.partial(
    pl.kernel,
    out_type=jax.ShapeDtypeStruct((NC, N_PAD, D), jnp.float32),
    mesh=plsc.VectorSubcoreMesh(core_axis_name="c", subcore_axis_name="s"),
    scratch_types=[
        pltpu.VMEM((C,), jnp.int32),
        pltpu.VMEM((C,), jnp.int32),
        pltpu.VMEM((C,), jnp.int32),
        pltpu.VMEM((C, D), jnp.float32),
        pltpu.VMEM((C, D), jnp.float32),
        pltpu.VMEM((C, D), jnp.float32),
        pltpu.VMEM((C, D), jnp.float32),
        pltpu.VMEM((D,), jnp.float32),
        pltpu.VMEM((ZR, D), jnp.float32),
        pltpu.VMEM_SHARED((N_PAD, D), jnp.float32),
        pltpu.SemaphoreType.DMA,
        pltpu.SemaphoreType.DMA,
        pltpu.SemaphoreType.DMA,
    ],
)
def _sc_edge_kernel(xa_hbm, xb_hbm, ec_hbm, row_hbm, col_hbm,
                    t_hbm, out_hbm,
                    rowva, rowvb, colv, bufa, bufb, bufea, bufeb, tbuf, zbuf,
                    s_acc, sem_a, sem_b, sem_e):
    cid = lax.axis_index("c")
    sid = lax.axis_index("s")
    wid = sid * NC + cid

    zvec = jnp.zeros((16,), jnp.float32)

    pltpu.sync_copy(t_hbm, tbuf)
    tvs = [tbuf[pl.ds(u * 16, 16)] for u in range(D // 16)]

    for zr in range(ZR):
        for u in range(D // 16):
            zbuf[zr, pl.ds(u * 16, 16)] = zvec

    def zero_stripe(k, _):
        pltpu.sync_copy(zbuf, s_acc.at[pl.ds(sid * RSTRIPE + k * ZR, ZR)])
        return 0
    lax.fori_loop(0, RSTRIPE // ZR, zero_stripe, 0)

    plsc.subcore_barrier()

    base_e = wid * EPW

    def fetch(i, rowv, bufe):
        off = base_e + i * C
        pltpu.sync_copy(row_hbm.at[pl.ds(off, C)], rowv)
        pltpu.sync_copy(col_hbm.at[pl.ds(off, C)], colv)
        pltpu.async_copy(xa_hbm.at[rowv], bufa, sem_a)
        pltpu.async_copy(xb_hbm.at[colv], bufb, sem_b)
        pltpu.async_copy(ec_hbm.at[pl.ds(off, C)], bufe, sem_e)

    def wait3(rowv, bufe):
        pltpu.make_async_copy(xa_hbm.at[rowv], bufa, sem_a).wait()
        pltpu.make_async_copy(xb_hbm.at[rowv], bufb, sem_b).wait()
        pltpu.make_async_copy(ec_hbm.at[pl.ds(0, C)], bufe, sem_e).wait()

    def comp(bufe):
        def comp1(r, _):
            for u in range(D // 16):
                cc = u * 16
                v = bufa[r, pl.ds(cc, 16)] + bufb[r, pl.ds(cc, 16)] \
                    + bufe[r, pl.ds(cc, 16)]
                bufe[r, pl.ds(cc, 16)] = (
                    jnp.maximum(v, jnp.float32(0.0)) + tvs[u])
            return 0
        lax.fori_loop(0, C, comp1, 0)

    def scat(rowv, bufe):
        plsc.subcore_barrier()

        def scat_ser(k, _):
            @pl.when(sid == k)
            def _():
                pltpu.sync_copy(bufe, s_acc.at[rowv], add=True)
            plsc.subcore_barrier()
            return 0
        lax.fori_loop(0, NS, scat_ser, 0)

    fetch(0, rowva, bufea)

    def pair(gi, _):
        i = 2 * gi
        wait3(rowva, bufea)
        comp(bufea)
        fetch(i + 1, rowvb, bufeb)
        scat(rowva, bufea)
        wait3(rowvb, bufeb)
        comp(bufeb)
        fetch(i + 2, rowva, bufea)
        scat(rowvb, bufeb)
        return 0
    lax.fori_loop(0, (NCHUNK - 1) // 2, pair, 0)

    wait3(rowva, bufea)
    comp(bufea)
    scat(rowva, bufea)

    plsc.subcore_barrier()

    def outcp(k, _):
        r0 = sid * RSTRIPE + k * C
        pltpu.sync_copy(s_acc.at[pl.ds(r0, C)], bufa)
        pltpu.sync_copy(bufa, out_hbm.at[cid, pl.ds(r0, C)])
        return 0
    lax.fori_loop(0, RSTRIPE // C, outcp, 0)


def kernel(x, edge_index, edge_attr, W1, b1, W2, b2):
    x = x.astype(jnp.float32)
    row = edge_index[0].astype(jnp.int32)
    col = edge_index[1].astype(jnp.int32)

    w1a = W1[:D]
    w1b = W1[D:2 * D]
    w1c = W1[2 * D:]

    t = jnp.linalg.solve(W2.T, b2)
    t = t + jnp.linalg.solve(W2.T, b2 - t @ W2)

    xa, xb = _proj_nodes(x, w1a, w1b)
    ec = _proj_edges(edge_attr, w1c, b1)

    s_parts = _sc_edge_kernel(xa, xb, ec, row, col, t)

    return _final_mm(s_parts, W2)[:N_NODES]
